# Initial kernel scaffold; baseline (speedup 1.0000x reference)
import jax
import jax.numpy as jnp
from jax.experimental import pallas as pl


def kernel(x_seq, edge_attr_seq, h_x, h_e, lap_vals, We, be, Wn, bn, coeff, edge_index, lap_rows, lap_cols):
    raise NotImplementedError("write your pallas kernel here")


# speedup vs baseline: 4.5342x; 4.5342x over previous
import functools

import jax
import jax.numpy as jnp
from jax import lax
from jax.experimental import pallas as pl
from jax.experimental.pallas import tpu as pltpu
from jax.experimental.pallas import tpu_sc as plsc

NC = 2
NS = 16
NW = NC * NS


def _make_sc_edge(n: int, e: int, de: int, chunk: int):
    ew = e // NW
    nrows = n // NS
    mesh = plsc.VectorSubcoreMesh(
        core_axis_name="c", subcore_axis_name="s", num_cores=NC,
        num_subcores=NS)

    @functools.partial(
        pl.kernel,
        out_type=(jax.ShapeDtypeStruct((e, de), jnp.float32),
                  jax.ShapeDtypeStruct((NC, n, de), jnp.float32)),
        mesh=mesh,
        scratch_types=[
            pltpu.VMEM((chunk,), jnp.int32),
            pltpu.VMEM((chunk,), jnp.int32),
            pltpu.VMEM((chunk, de), jnp.float32),
            pltpu.VMEM((chunk, de), jnp.float32),
            pltpu.VMEM((chunk, de), jnp.float32),
            pltpu.VMEM((nrows, de), jnp.float32),
            pltpu.VMEM_SHARED((n, de), jnp.float32),
            pltpu.VMEM_SHARED((n, de), jnp.float32),
            pltpu.VMEM_SHARED((n, de), jnp.float32),
            pltpu.SemaphoreType.DMA,
            pltpu.SemaphoreType.DMA,
        ],
        compiler_params=pltpu.CompilerParams(use_tc_tiling_on_sc=False),
    )
    def k(p_hbm, q_hbm, ep_hbm, src_hbm, dst_hbm, eout_hbm, agg_hbm,
          src_v, dst_v, pg_v, qg_v, ep_v, zb_v, p_sh, q_sh, acc_sh,
          sem1, sem2):
        cid = lax.axis_index("c")
        sid = lax.axis_index("s")
        wid = sid * NC + cid

        rr = pl.ds(sid * nrows, nrows)
        pltpu.sync_copy(p_hbm.at[rr], zb_v)
        pltpu.sync_copy(zb_v, p_sh.at[rr])
        pltpu.sync_copy(q_hbm.at[rr], zb_v)
        pltpu.sync_copy(zb_v, q_sh.at[rr])

        @pl.loop(0, nrows)
        def _(i):
            zb_v[i] = jnp.zeros((de,), jnp.float32)

        pltpu.sync_copy(zb_v, acc_sh.at[rr])
        plsc.subcore_barrier()

        base0 = wid * ew

        @pl.loop(0, ew // chunk)
        def _(kk):
            base = base0 + kk * chunk
            pltpu.sync_copy(src_hbm.at[pl.ds(base, chunk)], src_v)
            pltpu.sync_copy(dst_hbm.at[pl.ds(base, chunk)], dst_v)
            pltpu.sync_copy(ep_hbm.at[pl.ds(base, chunk)], ep_v)
            cp1 = pltpu.async_copy(p_sh.at[src_v], pg_v, sem1)
            cp2 = pltpu.async_copy(q_sh.at[dst_v], qg_v, sem2)
            cp1.wait()
            cp2.wait()

            @pl.loop(0, chunk)
            def _(i):
                ep_v[i] = jnp.maximum(pg_v[i] + qg_v[i] + ep_v[i], 0.0)

            pltpu.sync_copy(ep_v, eout_hbm.at[pl.ds(base, chunk)])
            pltpu.sync_copy(ep_v, acc_sh.at[dst_v], add=True)

        plsc.subcore_barrier()
        pltpu.sync_copy(acc_sh.at[pl.ds(sid * nrows, nrows)], zb_v)
        pltpu.sync_copy(zb_v, agg_hbm.at[cid, pl.ds(sid * nrows, nrows)])

    return k


def _make_sc_spmm(n: int, e: int, d: int, chunk: int):
    ew = e // NW
    nrows = n // NS
    zrows = nrows // 10
    mesh = plsc.VectorSubcoreMesh(
        core_axis_name="c", subcore_axis_name="s", num_cores=NC,
        num_subcores=NS)

    @functools.partial(
        pl.kernel,
        out_type=jax.ShapeDtypeStruct((NC, n, d), jnp.float32),
        mesh=mesh,
        scratch_types=[
            pltpu.VMEM((chunk,), jnp.int32),
            pltpu.VMEM((chunk,), jnp.int32),
            pltpu.VMEM((chunk,), jnp.float32),
            pltpu.VMEM((chunk, d), jnp.float32),
            pltpu.VMEM((zrows, d), jnp.float32),
            pltpu.VMEM_SHARED((n, d), jnp.float32),
            pltpu.SemaphoreType.DMA,
        ],
        compiler_params=pltpu.CompilerParams(use_tc_tiling_on_sc=False,
                                             needs_layout_passes=False),
    )
    def k(hx_hbm, cols_hbm, rows_hbm, vals_hbm, out_hbm,
          cols_v, rows_v, vals_v, g_v, zb_v, acc_sh, sem):
        cid = lax.axis_index("c")
        sid = lax.axis_index("s")
        wid = sid * NC + cid

        @pl.loop(0, zrows)
        def _(i):
            for j in range(d // 16):
                zb_v[i, pl.ds(j * 16, 16)] = jnp.zeros((16,), jnp.float32)

        for kz in range(nrows // zrows):
            pltpu.sync_copy(
                zb_v, acc_sh.at[pl.ds(sid * nrows + kz * zrows, zrows)])
        plsc.subcore_barrier()

        base0 = wid * ew

        @pl.loop(0, ew // chunk)
        def _(kk):
            base = base0 + kk * chunk
            pltpu.sync_copy(cols_hbm.at[pl.ds(base, chunk)], cols_v)
            pltpu.sync_copy(rows_hbm.at[pl.ds(base, chunk)], rows_v)
            pltpu.sync_copy(vals_hbm.at[pl.ds(base, chunk)], vals_v)
            pltpu.async_copy(hx_hbm.at[cols_v], g_v, sem).wait()

            @pl.loop(0, chunk)
            def _(i):
                s = plsc.load_gather(vals_v, [jnp.full((16,), i, jnp.int32)])
                for j in range(d // 16):
                    g_v[i, pl.ds(j * 16, 16)] = g_v[i, pl.ds(j * 16, 16)] * s

            pltpu.sync_copy(g_v, acc_sh.at[rows_v], add=True)

        plsc.subcore_barrier()
        for kz in range(nrows // zrows):
            pltpu.sync_copy(
                acc_sh.at[pl.ds(sid * nrows + kz * zrows, zrows)], zb_v)
            pltpu.sync_copy(
                zb_v, out_hbm.at[cid, pl.ds(sid * nrows + kz * zrows, zrows)])

    return k


def _tc_pre_node(x_ref, hx_ref, wpq_ref, wnx_ref, bn_ref,
                 p_ref, q_ref, s1_ref, *, d, de):
    x = x_ref[...]
    h = hx_ref[...]
    pq = (jnp.dot(x, wpq_ref[0:d], preferred_element_type=jnp.float32)
          + jnp.dot(h, wpq_ref[d:2 * d], preferred_element_type=jnp.float32))
    p_ref[...] = pq[:, 0:de]
    q_ref[...] = pq[:, de:2 * de]
    s1_ref[...] = (jnp.dot(x, wnx_ref[0:d], preferred_element_type=jnp.float32)
                   + jnp.dot(h, wnx_ref[d:2 * d],
                             preferred_element_type=jnp.float32)
                   + bn_ref[...])


def _tc_edge_pre(ea_ref, he_ref, wee_ref, be_ref, ep_ref, *, de):
    ep_ref[...] = (
        jnp.dot(ea_ref[...], wee_ref[0:de], preferred_element_type=jnp.float32)
        + jnp.dot(he_ref[...], wee_ref[de:2 * de],
                  preferred_element_type=jnp.float32)
        + be_ref[...])


def _tc_post_node(s1_ref, a0_ref, a1_ref, hx_ref, sp0_ref, sp1_ref, wna_ref,
                  coeff_ref, xo_ref, td_ref, sp_ref):
    agg = a0_ref[...] + a1_ref[...]
    xo = s1_ref[...] + jnp.dot(agg, wna_ref[...],
                               preferred_element_type=jnp.float32)
    xo_ref[...] = xo
    td_ref[...] = xo - hx_ref[...]
    sp_ref[...] = (-coeff_ref[0, 0]) * (sp0_ref[...] + sp1_ref[...])


def kernel(x_seq, edge_attr_seq, h_x, h_e, lap_vals, We, be, Wn, bn, coeff,
           edge_index, lap_rows, lap_cols):
    t_steps, n, d = x_seq.shape
    e, de = edge_attr_seq.shape[1], edge_attr_seq.shape[2]

    src = edge_index[0]
    dst = edge_index[1]
    wpq = jnp.concatenate([We[0:2 * d], We[2 * d:4 * d]], axis=1)
    wee = We[4 * d:]
    wnx = Wn[0:2 * d]
    wna = Wn[2 * d:]
    be2 = be.reshape(1, de)
    bn2 = bn.reshape(1, d)
    coeff2 = jnp.reshape(coeff, (1, 1))

    bn_blk = 2000
    be_blk = 16000

    pre_node = pl.pallas_call(
        functools.partial(_tc_pre_node, d=d, de=de),
        grid=(n // bn_blk,),
        in_specs=[
            pl.BlockSpec((bn_blk, d), lambda i: (i, 0)),
            pl.BlockSpec((bn_blk, d), lambda i: (i, 0)),
            pl.BlockSpec((2 * d, 2 * de), lambda i: (0, 0)),
            pl.BlockSpec((2 * d, d), lambda i: (0, 0)),
            pl.BlockSpec((1, d), lambda i: (0, 0)),
        ],
        out_specs=[
            pl.BlockSpec((bn_blk, de), lambda i: (i, 0)),
            pl.BlockSpec((bn_blk, de), lambda i: (i, 0)),
            pl.BlockSpec((bn_blk, d), lambda i: (i, 0)),
        ],
        out_shape=[
            jax.ShapeDtypeStruct((n, de), jnp.float32),
            jax.ShapeDtypeStruct((n, de), jnp.float32),
            jax.ShapeDtypeStruct((n, d), jnp.float32),
        ],
    )

    edge_pre = pl.pallas_call(
        functools.partial(_tc_edge_pre, de=de),
        grid=(e // be_blk,),
        in_specs=[
            pl.BlockSpec((be_blk, de), lambda i: (i, 0)),
            pl.BlockSpec((be_blk, de), lambda i: (i, 0)),
            pl.BlockSpec((2 * de, de), lambda i: (0, 0)),
            pl.BlockSpec((1, de), lambda i: (0, 0)),
        ],
        out_specs=pl.BlockSpec((be_blk, de), lambda i: (i, 0)),
        out_shape=jax.ShapeDtypeStruct((e, de), jnp.float32),
    )

    post_node = pl.pallas_call(
        _tc_post_node,
        grid=(n // bn_blk,),
        in_specs=[
            pl.BlockSpec((bn_blk, d), lambda i: (i, 0)),
            pl.BlockSpec((bn_blk, de), lambda i: (i, 0)),
            pl.BlockSpec((bn_blk, de), lambda i: (i, 0)),
            pl.BlockSpec((bn_blk, d), lambda i: (i, 0)),
            pl.BlockSpec((bn_blk, d), lambda i: (i, 0)),
            pl.BlockSpec((bn_blk, d), lambda i: (i, 0)),
            pl.BlockSpec((de, d), lambda i: (0, 0)),
            pl.BlockSpec(memory_space=pltpu.SMEM),
        ],
        out_specs=[
            pl.BlockSpec((bn_blk, d), lambda i: (i, 0)),
            pl.BlockSpec((bn_blk, d), lambda i: (i, 0)),
            pl.BlockSpec((bn_blk, d), lambda i: (i, 0)),
        ],
        out_shape=[
            jax.ShapeDtypeStruct((n, d), jnp.float32),
            jax.ShapeDtypeStruct((n, d), jnp.float32),
            jax.ShapeDtypeStruct((n, d), jnp.float32),
        ],
    )

    n_pad = ((n + 639) // 640) * 640
    sc_edge = _make_sc_edge(n_pad, e, de, chunk=1000)
    sc_spmm = _make_sc_spmm(n_pad, e, d, chunk=200)

    hx, he = h_x, h_e
    out_x, out_e, tds, sps = [], [], [], []
    for t in range(t_steps):
        p, q, s1 = pre_node(x_seq[t], hx, wpq, wnx, bn2)
        p = jnp.pad(p, ((0, n_pad - n), (0, 0)))
        q = jnp.pad(q, ((0, n_pad - n), (0, 0)))
        epart = edge_pre(edge_attr_seq[t], he, wee, be2)
        e_out, agg2 = sc_edge(p, q, epart, src, dst)
        sp2 = sc_spmm(hx, lap_cols, lap_rows, lap_vals)
        x_out, td, sp = post_node(s1, agg2[0, :n], agg2[1, :n], hx,
                                  sp2[0, :n], sp2[1, :n], wna, coeff2)
        hx, he = x_out, e_out
        out_x.append(x_out)
        out_e.append(e_out)
        tds.append(td)
        sps.append(sp)

    return (jnp.stack(out_x), jnp.stack(out_e), jnp.stack(tds),
            jnp.stack(sps))
